# Initial kernel scaffold; baseline (speedup 1.0000x reference)
import functools

import jax
import jax.numpy as jnp
from jax import lax
from jax.experimental import pallas as pl
from jax.experimental.pallas import tpu as pltpu
from jax.experimental.pallas import tpu_sc as plsc

N_CORES = 2
N_SUB = 16
NW = N_CORES * N_SUB

N = 10000
E = 320000
D = 128
DE = 16

W = 80
E_PER_TILE = E // NW
CHUNKS = E_PER_TILE // W
N_PER_SUB = N // N_SUB


def _proj_body(x_ref, we_ref, xs_ref, xd_ref):
    xb = x_ref[...]
    we = we_ref[...]
    xs_ref[...] = jnp.dot(xb, we[0:D, :], preferred_element_type=jnp.float32)
    xd_ref[...] = jnp.dot(xb, we[D:2 * D, :], preferred_element_type=jnp.float32)


def _node_proj(x2d, W_e):
    blk = 1000
    return pl.pallas_call(
        _proj_body,
        grid=(N // blk,),
        in_specs=[
            pl.BlockSpec((blk, D), lambda i: (i, 0)),
            pl.BlockSpec((2 * D + DE, DE), lambda i: (0, 0)),
        ],
        out_specs=[
            pl.BlockSpec((blk, DE), lambda i: (i, 0)),
            pl.BlockSpec((blk, DE), lambda i: (i, 0)),
        ],
        out_shape=[
            jax.ShapeDtypeStruct((N, DE), jnp.float32),
            jax.ShapeDtypeStruct((N, DE), jnp.float32),
        ],
    )(x2d, W_e)


def _attr_body(a_ref, wa_ref, be_ref, out_ref):
    out_ref[...] = (
        jnp.dot(a_ref[...], wa_ref[...], preferred_element_type=jnp.float32)
        + be_ref[...]
    )


def _attr_proj(attr2d, W_attr, b_e2d):
    blk = 2000
    return pl.pallas_call(
        _attr_body,
        grid=(E // blk,),
        in_specs=[
            pl.BlockSpec((blk, DE), lambda i: (i, 0)),
            pl.BlockSpec((DE, DE), lambda i: (0, 0)),
            pl.BlockSpec((1, DE), lambda i: (0, 0)),
        ],
        out_specs=pl.BlockSpec((blk, DE), lambda i: (i, 0)),
        out_shape=jax.ShapeDtypeStruct((E, DE), jnp.float32),
    )(attr2d, W_attr, b_e2d)


def _sc_edge_kernel(xs2d, xd2d, row, col, attr_p):
    mesh = plsc.VectorSubcoreMesh(core_axis_name="c", subcore_axis_name="s")

    @functools.partial(
        pl.kernel,
        out_type=(
            jax.ShapeDtypeStruct((E, DE), jnp.float32),
            jax.ShapeDtypeStruct((N_CORES, N, DE), jnp.float32),
        ),
        mesh=mesh,
        scratch_types=[
            pltpu.VMEM((W,), jnp.int32),
            pltpu.VMEM((W,), jnp.int32),
            pltpu.VMEM((W, DE), jnp.float32),
            pltpu.VMEM((W, DE), jnp.float32),
            pltpu.VMEM((W, DE), jnp.float32),
            pltpu.VMEM((W, DE), jnp.float32),
            pltpu.VMEM((N_PER_SUB, DE), jnp.float32),
            pltpu.VMEM_SHARED((N, DE), jnp.float32),
            pltpu.SemaphoreType.DMA,
        ],
    )
    def k(xs_hbm, xd_hbm, row_hbm, col_hbm, attr_hbm, ea_hbm, agg_hbm,
          row_v, col_v, src_v, dst_v, attr_v, ea_v, zbuf, agg_sp, sem):
        c = lax.axis_index("c")
        s = lax.axis_index("s")
        wid = s * N_CORES + c

        @pl.loop(0, N_PER_SUB)
        def _(i):
            zbuf[i, :] = jnp.zeros((DE,), jnp.float32)

        nrows = pl.ds(s * N_PER_SUB, N_PER_SUB)
        pltpu.sync_copy(zbuf, agg_sp.at[nrows])
        plsc.subcore_barrier()

        @pl.loop(0, CHUNKS)
        def _(j):
            off = wid * E_PER_TILE + j * W
            esl = pl.ds(off, W)
            pltpu.sync_copy(row_hbm.at[esl], row_v)
            pltpu.sync_copy(col_hbm.at[esl], col_v)
            pltpu.sync_copy(attr_hbm.at[esl], attr_v)
            pltpu.async_copy(xs_hbm.at[row_v], src_v, sem).wait()
            pltpu.async_copy(xd_hbm.at[col_v], dst_v, sem).wait()

            @pl.loop(0, W)
            def _(i):
                ea_v[i, :] = jnp.maximum(
                    src_v[i, :] + dst_v[i, :] + attr_v[i, :], 0.0)

            pltpu.sync_copy(ea_v, ea_hbm.at[esl])
            pltpu.sync_copy(ea_v, agg_sp.at[col_v], add=True)

        plsc.subcore_barrier()
        pltpu.sync_copy(agg_sp.at[nrows], zbuf)
        pltpu.sync_copy(zbuf, agg_hbm.at[c].at[nrows])

    return k(xs2d, xd2d, row, col, attr_p)


def _node_body(x_ref, agg_ref, wn_ref, bn_ref, out_ref):
    wn = wn_ref[...]
    agg = agg_ref[0] + agg_ref[1]
    out_ref[...] = (
        jnp.dot(x_ref[...], wn[0:D, :], preferred_element_type=jnp.float32)
        + jnp.dot(agg, wn[D:D + DE, :], preferred_element_type=jnp.float32)
        + bn_ref[...]
    )


def _node_update(x2d, agg, W_n, b_n2d):
    blk = 1000
    return pl.pallas_call(
        _node_body,
        grid=(N // blk,),
        in_specs=[
            pl.BlockSpec((blk, D), lambda i: (i, 0)),
            pl.BlockSpec((N_CORES, blk, DE), lambda i: (0, i, 0)),
            pl.BlockSpec((D + DE, D), lambda i: (0, 0)),
            pl.BlockSpec((1, D), lambda i: (0, 0)),
        ],
        out_specs=pl.BlockSpec((blk, D), lambda i: (i, 0)),
        out_shape=jax.ShapeDtypeStruct((N, D), jnp.float32),
    )(x2d, agg, W_n, b_n2d)


def kernel(x, edge_index, edge_attr, W_e, b_e, W_n, b_n):
    x2d = x[0]
    row = edge_index[0, 0]
    col = edge_index[0, 1]
    attr2d = edge_attr[0]

    xs2d, xd2d = _node_proj(x2d, W_e)
    attr_p = _attr_proj(attr2d, W_e[2 * D:], b_e.reshape(1, DE))
    ea, agg = _sc_edge_kernel(xs2d, xd2d, row, col, attr_p)
    x_new = _node_update(x2d, agg, W_n, b_n.reshape(1, D))
    return (x_new[None], ea[None])


# speedup vs baseline: 7.7643x; 7.7643x over previous
import functools

import jax
import jax.numpy as jnp
from jax import lax
from jax.experimental import pallas as pl
from jax.experimental.pallas import tpu as pltpu
from jax.experimental.pallas import tpu_sc as plsc

N_CORES = 2
N_SUB = 16
NW = N_CORES * N_SUB

N = 10000
E = 320000
D = 128
DE = 16

W = 80
E_PER_TILE = E // NW
CHUNKS = E_PER_TILE // W
N_PAD = 10240
N_PER_SUB = N_PAD // N_SUB


def _proj_body(x_ref, we_ref, xs_ref, xd_ref):
    xb = x_ref[...]
    we = we_ref[...]
    xs_ref[...] = jnp.dot(xb, we[0:D, :], preferred_element_type=jnp.float32)
    xd_ref[...] = jnp.dot(xb, we[D:2 * D, :], preferred_element_type=jnp.float32)


def _node_proj(x2d, W_e):
    blk = 1000
    return pl.pallas_call(
        _proj_body,
        grid=(N // blk,),
        in_specs=[
            pl.BlockSpec((blk, D), lambda i: (i, 0)),
            pl.BlockSpec((2 * D + DE, DE), lambda i: (0, 0)),
        ],
        out_specs=[
            pl.BlockSpec((blk, DE), lambda i: (i, 0)),
            pl.BlockSpec((blk, DE), lambda i: (i, 0)),
        ],
        out_shape=[
            jax.ShapeDtypeStruct((N, DE), jnp.float32),
            jax.ShapeDtypeStruct((N, DE), jnp.float32),
        ],
    )(x2d, W_e)


def _attr_body(a_ref, wa_ref, be_ref, out_ref):
    out_ref[...] = (
        jnp.dot(a_ref[...], wa_ref[...], preferred_element_type=jnp.float32)
        + be_ref[...]
    )


def _attr_proj(attr2d, W_attr, b_e2d):
    blk = 2000
    return pl.pallas_call(
        _attr_body,
        grid=(E // blk,),
        in_specs=[
            pl.BlockSpec((blk, DE), lambda i: (i, 0)),
            pl.BlockSpec((DE, DE), lambda i: (0, 0)),
            pl.BlockSpec((1, DE), lambda i: (0, 0)),
        ],
        out_specs=pl.BlockSpec((blk, DE), lambda i: (i, 0)),
        out_shape=jax.ShapeDtypeStruct((E, DE), jnp.float32),
    )(attr2d, W_attr, b_e2d)


def _sc_edge_kernel(xs2d, xd2d, row, col, attr_p):
    mesh = plsc.VectorSubcoreMesh(core_axis_name="c", subcore_axis_name="s")

    @functools.partial(
        pl.kernel,
        out_type=(
            jax.ShapeDtypeStruct((E, DE), jnp.float32),
            jax.ShapeDtypeStruct((N_CORES, N_PAD, DE), jnp.float32),
        ),
        mesh=mesh,
        scratch_types=[
            pltpu.VMEM((W,), jnp.int32),
            pltpu.VMEM((W,), jnp.int32),
            pltpu.VMEM((W, DE), jnp.float32),
            pltpu.VMEM((W, DE), jnp.float32),
            pltpu.VMEM((W, DE), jnp.float32),
            pltpu.VMEM((W, DE), jnp.float32),
            pltpu.VMEM((N_PER_SUB, DE), jnp.float32),
            pltpu.VMEM_SHARED((N_PAD, DE), jnp.float32),
            pltpu.SemaphoreType.DMA,
        ],
        compiler_params=pltpu.CompilerParams(use_tc_tiling_on_sc=False),
    )
    def k(xs_hbm, xd_hbm, row_hbm, col_hbm, attr_hbm, ea_hbm, agg_hbm,
          row_v, col_v, src_v, dst_v, attr_v, ea_v, zbuf, agg_sp, sem):
        c = lax.axis_index("c")
        s = lax.axis_index("s")
        wid = s * N_CORES + c

        @pl.loop(0, N_PER_SUB)
        def _(i):
            zbuf[i, :] = jnp.zeros((DE,), jnp.float32)

        nrows = pl.ds(s * N_PER_SUB, N_PER_SUB)
        pltpu.sync_copy(zbuf, agg_sp.at[nrows])
        plsc.subcore_barrier()

        @pl.loop(0, CHUNKS)
        def _(j):
            off = wid * E_PER_TILE + j * W
            esl = pl.ds(off, W)
            pltpu.sync_copy(row_hbm.at[esl], row_v)
            pltpu.sync_copy(col_hbm.at[esl], col_v)
            pltpu.sync_copy(attr_hbm.at[esl], attr_v)
            pltpu.async_copy(xs_hbm.at[row_v], src_v, sem).wait()
            pltpu.async_copy(xd_hbm.at[col_v], dst_v, sem).wait()

            @pl.loop(0, W)
            def _(i):
                ea_v[i, :] = jnp.maximum(
                    src_v[i, :] + dst_v[i, :] + attr_v[i, :], 0.0)

            pltpu.sync_copy(ea_v, ea_hbm.at[esl])
            pltpu.sync_copy(ea_v, agg_sp.at[col_v], add=True)

        plsc.subcore_barrier()
        pltpu.sync_copy(agg_sp.at[nrows], zbuf)
        pltpu.sync_copy(zbuf, agg_hbm.at[c].at[nrows])

    return k(xs2d, xd2d, row, col, attr_p)


def _node_body(x_ref, agg_ref, wn_ref, bn_ref, out_ref):
    wn = wn_ref[...]
    agg = agg_ref[0] + agg_ref[1]
    out_ref[...] = (
        jnp.dot(x_ref[...], wn[0:D, :], preferred_element_type=jnp.float32)
        + jnp.dot(agg, wn[D:D + DE, :], preferred_element_type=jnp.float32)
        + bn_ref[...]
    )


def _node_update(x2d, agg, W_n, b_n2d):
    blk = 1000
    return pl.pallas_call(
        _node_body,
        grid=(N // blk,),
        in_specs=[
            pl.BlockSpec((blk, D), lambda i: (i, 0)),
            pl.BlockSpec((N_CORES, blk, DE), lambda i: (0, i, 0)),
            pl.BlockSpec((D + DE, D), lambda i: (0, 0)),
            pl.BlockSpec((1, D), lambda i: (0, 0)),
        ],
        out_specs=pl.BlockSpec((blk, D), lambda i: (i, 0)),
        out_shape=jax.ShapeDtypeStruct((N, D), jnp.float32),
    )(x2d, agg, W_n, b_n2d)


def kernel(x, edge_index, edge_attr, W_e, b_e, W_n, b_n):
    x2d = x[0]
    row = edge_index[0, 0]
    col = edge_index[0, 1]
    attr2d = edge_attr[0]

    xs2d, xd2d = _node_proj(x2d, W_e)
    attr_p = _attr_proj(attr2d, W_e[2 * D:], b_e.reshape(1, DE))
    ea, agg = _sc_edge_kernel(xs2d, xd2d, row, col, attr_p)
    x_new = _node_update(x2d, agg, W_n, b_n.reshape(1, D))
    return (x_new[None], ea[None])


# speedup vs baseline: 17.1452x; 2.2082x over previous
import functools

import jax
import jax.numpy as jnp
from jax import lax
from jax.experimental import pallas as pl
from jax.experimental.pallas import tpu as pltpu
from jax.experimental.pallas import tpu_sc as plsc

N_CORES = 2
N_SUB = 16
NW = N_CORES * N_SUB

N = 10000
E = 320000
D = 128
DE = 16

W = 80
SUP = 2000
CPS = SUP // W
NSUP = (E // NW) // SUP
EW = E // 8
N_PAD = 10240
N_PER_SUB = N_PAD // N_SUB


def _pre_body(x_ref, aw_ref, we_ref, wb_ref, bb_ref, xs_ref, xd_ref, ap_ref):
    xb = x_ref[...]
    we = we_ref[...]
    xs_ref[...] = jnp.dot(xb, we[0:D, :], preferred_element_type=jnp.float32)
    xd_ref[...] = jnp.dot(xb, we[D:2 * D, :], preferred_element_type=jnp.float32)
    ap_ref[...] = (
        jnp.dot(aw_ref[...], wb_ref[...], preferred_element_type=jnp.float32)
        + bb_ref[...]
    )


def _pre(x2d, attr_w, W_e, W_big, b_big):
    nblk = 1000
    ablk = EW // (N // nblk)
    return pl.pallas_call(
        _pre_body,
        grid=(N // nblk,),
        in_specs=[
            pl.BlockSpec((nblk, D), lambda i: (i, 0)),
            pl.BlockSpec((ablk, D), lambda i: (i, 0)),
            pl.BlockSpec((2 * D + DE, DE), lambda i: (0, 0)),
            pl.BlockSpec((D, D), lambda i: (0, 0)),
            pl.BlockSpec((1, D), lambda i: (0, 0)),
        ],
        out_specs=[
            pl.BlockSpec((nblk, DE), lambda i: (i, 0)),
            pl.BlockSpec((nblk, DE), lambda i: (i, 0)),
            pl.BlockSpec((ablk, D), lambda i: (i, 0)),
        ],
        out_shape=[
            jax.ShapeDtypeStruct((N, DE), jnp.float32),
            jax.ShapeDtypeStruct((N, DE), jnp.float32),
            jax.ShapeDtypeStruct((EW, D), jnp.float32),
        ],
    )(x2d, attr_w, W_e, W_big, b_big)


def _sc_edge_kernel(xs2d, xd2d, row2, col2, attr_pw):
    mesh = plsc.VectorSubcoreMesh(core_axis_name="c", subcore_axis_name="s")

    @functools.partial(
        pl.kernel,
        out_type=(
            jax.ShapeDtypeStruct((E, DE), jnp.float32),
            jax.ShapeDtypeStruct((N_CORES, N_PAD, DE), jnp.float32),
        ),
        mesh=mesh,
        scratch_types=[
            pltpu.VMEM((CPS, W), jnp.int32),
            pltpu.VMEM((CPS, W), jnp.int32),
            pltpu.VMEM((SUP, DE), jnp.float32),
            pltpu.VMEM((SUP, DE), jnp.float32),
            pltpu.VMEM((SUP // 8, D), jnp.float32),
            pltpu.VMEM((N_PER_SUB, DE), jnp.float32),
            pltpu.VMEM_SHARED((N_PAD, DE), jnp.float32),
            pltpu.SemaphoreType.DMA,
            pltpu.SemaphoreType.DMA,
            pltpu.SemaphoreType.DMA,
        ],
        compiler_params=pltpu.CompilerParams(use_tc_tiling_on_sc=False),
    )
    def k(xs_hbm, xd_hbm, row_hbm, col_hbm, attr_hbm, ea_hbm, agg_hbm,
          row_v, col_v, src_v, dst_v, attr_v, zbuf, agg_sp,
          isem, gsem, osem):
        c = lax.axis_index("c")
        s = lax.axis_index("s")
        wid = s * N_CORES + c

        @pl.loop(0, N_PER_SUB)
        def _(i):
            zbuf[i, :] = jnp.zeros((DE,), jnp.float32)

        nrows = pl.ds(s * N_PER_SUB, N_PER_SUB)
        pltpu.sync_copy(zbuf, agg_sp.at[nrows])
        plsc.subcore_barrier()

        @pl.loop(0, NSUP)
        def _(ss):
            erow = wid * (NSUP * CPS) + ss * CPS
            arow = wid * (NSUP * SUP // 8) + ss * (SUP // 8)
            eoff = wid * (NSUP * SUP) + ss * SUP

            i1 = pltpu.async_copy(row_hbm.at[pl.ds(erow, CPS)], row_v, isem)
            i2 = pltpu.async_copy(col_hbm.at[pl.ds(erow, CPS)], col_v, isem)
            i3 = pltpu.async_copy(
                attr_hbm.at[pl.ds(arow, SUP // 8)], attr_v, isem)
            i1.wait(); i2.wait(); i3.wait()

            @pl.loop(0, 5)
            def _(g):
                for tt in range(5):
                    t = g * 5 + tt
                    dsl = pl.ds(t * W, W)
                    pltpu.async_copy(
                        xs_hbm.at[row_v.at[t]], src_v.at[dsl], gsem)
                    pltpu.async_copy(
                        xd_hbm.at[col_v.at[t]], dst_v.at[dsl], gsem)
                for tt in range(5):
                    t = g * 5 + tt
                    dsl = pl.ds(t * W, W)
                    pltpu.make_async_copy(
                        xs_hbm.at[row_v.at[t]], src_v.at[dsl], gsem).wait()
                    pltpu.make_async_copy(
                        xd_hbm.at[col_v.at[t]], dst_v.at[dsl], gsem).wait()

            @pl.loop(0, SUP // 8)
            def _(r):
                for kk in range(8):
                    i = r * 8 + kk
                    src_v[i, :] = jnp.maximum(
                        src_v[i, :] + dst_v[i, :]
                        + attr_v[r, pl.ds(kk * DE, DE)], 0.0)

            o1 = pltpu.async_copy(src_v, ea_hbm.at[pl.ds(eoff, SUP)], osem)

            @pl.loop(0, CPS)
            def _(t):
                dsl = pl.ds(t * W, W)
                pltpu.sync_copy(
                    src_v.at[dsl], agg_sp.at[col_v.at[t]], add=True)

            o1.wait()

        plsc.subcore_barrier()
        pltpu.sync_copy(agg_sp.at[nrows], zbuf)
        pltpu.sync_copy(zbuf, agg_hbm.at[c].at[nrows])

    return k(xs2d, xd2d, row2, col2, attr_pw)


def _node_body(x_ref, agg_ref, wn_ref, bn_ref, out_ref):
    wn = wn_ref[...]
    agg = agg_ref[0] + agg_ref[1]
    out_ref[...] = (
        jnp.dot(x_ref[...], wn[0:D, :], preferred_element_type=jnp.float32)
        + jnp.dot(agg, wn[D:D + DE, :], preferred_element_type=jnp.float32)
        + bn_ref[...]
    )


def _node_update(x2d, agg, W_n, b_n2d):
    blk = 1000
    return pl.pallas_call(
        _node_body,
        grid=(N // blk,),
        in_specs=[
            pl.BlockSpec((blk, D), lambda i: (i, 0)),
            pl.BlockSpec((N_CORES, blk, DE), lambda i: (0, i, 0)),
            pl.BlockSpec((D + DE, D), lambda i: (0, 0)),
            pl.BlockSpec((1, D), lambda i: (0, 0)),
        ],
        out_specs=pl.BlockSpec((blk, D), lambda i: (i, 0)),
        out_shape=jax.ShapeDtypeStruct((N, D), jnp.float32),
    )(x2d, agg, W_n, b_n2d)


def kernel(x, edge_index, edge_attr, W_e, b_e, W_n, b_n):
    x2d = x[0]
    row2 = edge_index[0, 0].reshape(E // W, W)
    col2 = edge_index[0, 1].reshape(E // W, W)
    attr_w = edge_attr.reshape(EW, D)

    W_attr = W_e[2 * D:]
    W_big = jnp.kron(jnp.eye(8, dtype=jnp.float32), W_attr)
    b_big = jnp.tile(b_e, 8).reshape(1, D)

    xs2d, xd2d, attr_pw = _pre(x2d, attr_w, W_e, W_big, b_big)
    ea, agg = _sc_edge_kernel(xs2d, xd2d, row2, col2, attr_pw)
    x_new = _node_update(x2d, agg, W_n, b_n.reshape(1, D))
    return (x_new[None], ea[None])


# speedup vs baseline: 22.6948x; 1.3237x over previous
import functools

import jax
import jax.numpy as jnp
from jax import lax
from jax.experimental import pallas as pl
from jax.experimental.pallas import tpu as pltpu
from jax.experimental.pallas import tpu_sc as plsc

N_CORES = 2
N_SUB = 16
NW = N_CORES * N_SUB

N = 10000
E = 320000
D = 128
DE = 16

W = 40
SUP = 1000
CPS = SUP // W
NSUP = (E // NW) // SUP
N_PAD = 10240
N_PER_SUB = N_PAD // N_SUB
ZB = 64


def _pre_body(x_ref, at_ref, we_ref, be_ref, xs_ref, xd_ref, apt_ref):
    xb = x_ref[...]
    we = we_ref[...]
    xs_ref[...] = jnp.dot(xb, we[0:D, :], preferred_element_type=jnp.float32)
    xd_ref[...] = jnp.dot(xb, we[D:2 * D, :], preferred_element_type=jnp.float32)
    apt_ref[...] = lax.dot_general(
        we[2 * D:, :], at_ref[...],
        dimension_numbers=(((0,), (0,)), ((), ())),
        preferred_element_type=jnp.float32,
    ) + be_ref[...]


def _pre(x2d, attr_t, W_e, b_e_col):
    nblk = 1000
    ablk = E // (N // nblk)
    return pl.pallas_call(
        _pre_body,
        grid=(N // nblk,),
        in_specs=[
            pl.BlockSpec((nblk, D), lambda i: (i, 0)),
            pl.BlockSpec((DE, ablk), lambda i: (0, i)),
            pl.BlockSpec((2 * D + DE, DE), lambda i: (0, 0)),
            pl.BlockSpec((DE, 1), lambda i: (0, 0)),
        ],
        out_specs=[
            pl.BlockSpec((nblk, DE), lambda i: (i, 0)),
            pl.BlockSpec((nblk, DE), lambda i: (i, 0)),
            pl.BlockSpec((DE, ablk), lambda i: (0, i)),
        ],
        out_shape=[
            jax.ShapeDtypeStruct((N, DE), jnp.float32),
            jax.ShapeDtypeStruct((N, DE), jnp.float32),
            jax.ShapeDtypeStruct((DE, E), jnp.float32),
        ],
    )(x2d, attr_t, W_e, b_e_col)


def _sc_edge_kernel(xs2d, xd2d, row2, col2, attr_pt):
    mesh = plsc.VectorSubcoreMesh(core_axis_name="c", subcore_axis_name="s")

    @functools.partial(
        pl.kernel,
        out_type=(
            jax.ShapeDtypeStruct((DE, E), jnp.float32),
            jax.ShapeDtypeStruct((N_CORES, N_PAD, DE), jnp.float32),
        ),
        mesh=mesh,
        scratch_types=[
            pltpu.VMEM((CPS, W), jnp.int32),
            pltpu.VMEM((CPS, W), jnp.int32),
            pltpu.VMEM((SUP, DE), jnp.float32),
            pltpu.VMEM((SUP, DE), jnp.float32),
            pltpu.VMEM((DE, SUP), jnp.float32),
            pltpu.VMEM((DE, SUP), jnp.float32),
            pltpu.VMEM((ZB, DE), jnp.float32),
            pltpu.VMEM_SHARED((N_PAD, DE), jnp.float32),
            pltpu.SemaphoreType.DMA,
            pltpu.SemaphoreType.DMA,
            pltpu.SemaphoreType.DMA,
        ],
        compiler_params=pltpu.CompilerParams(
            use_tc_tiling_on_sc=False, needs_layout_passes=False),
    )
    def k(xs_hbm, xd_hbm, row_hbm, col_hbm, attr_hbm, ea_hbm, agg_hbm,
          row_v, col_v, src_v, dst_v, attr_v, eat_v, zbuf, agg_sp,
          isem, gsem, osem):
        c = lax.axis_index("c")
        s = lax.axis_index("s")
        wid = s * N_CORES + c
        lane = lax.iota(jnp.int32, DE)

        @pl.loop(0, ZB)
        def _(i):
            zbuf[i, :] = jnp.zeros((DE,), jnp.float32)

        @pl.loop(0, N_PER_SUB // ZB)
        def _(j):
            pltpu.sync_copy(
                zbuf, agg_sp.at[pl.ds(s * N_PER_SUB + j * ZB, ZB)])

        plsc.subcore_barrier()

        @pl.loop(0, NSUP)
        def _(ss):
            eoff = wid * (NSUP * SUP) + ss * SUP
            erow = eoff // W

            i1 = pltpu.async_copy(row_hbm.at[pl.ds(erow, CPS)], row_v, isem)
            i2 = pltpu.async_copy(col_hbm.at[pl.ds(erow, CPS)], col_v, isem)
            i3 = pltpu.async_copy(
                attr_hbm.at[pl.ds(0, DE), pl.ds(eoff, SUP)], attr_v, isem)
            i1.wait(); i2.wait(); i3.wait()

            @pl.loop(0, 5)
            def _(g):
                for tt in range(5):
                    t = g * 5 + tt
                    dsl = pl.ds(t * W, W)
                    pltpu.async_copy(
                        xs_hbm.at[row_v.at[t]], src_v.at[dsl], gsem)
                    pltpu.async_copy(
                        xd_hbm.at[col_v.at[t]], dst_v.at[dsl], gsem)
                for tt in range(5):
                    t = g * 5 + tt
                    dsl = pl.ds(t * W, W)
                    pltpu.make_async_copy(
                        xs_hbm.at[row_v.at[t]], src_v.at[dsl], gsem).wait()
                    pltpu.make_async_copy(
                        xd_hbm.at[col_v.at[t]], dst_v.at[dsl], gsem).wait()

            @pl.loop(0, SUP)
            def _(i):
                icol = jnp.full((DE,), i, jnp.int32)
                av = plsc.load_gather(attr_v, [lane, icol])
                ea = jnp.maximum(src_v[i, :] + dst_v[i, :] + av, 0.0)
                src_v[i, :] = ea
                plsc.store_scatter(eat_v, [lane, icol], ea)

            o1 = pltpu.async_copy(
                eat_v, ea_hbm.at[pl.ds(0, DE), pl.ds(eoff, SUP)], osem)

            @pl.loop(0, CPS)
            def _(t):
                dsl = pl.ds(t * W, W)
                pltpu.sync_copy(
                    src_v.at[dsl], agg_sp.at[col_v.at[t]], add=True)

            o1.wait()

        plsc.subcore_barrier()
        @pl.loop(0, N_PER_SUB // ZB)
        def _(j):
            nsl = pl.ds(s * N_PER_SUB + j * ZB, ZB)
            pltpu.sync_copy(agg_sp.at[nsl], zbuf)
            pltpu.sync_copy(zbuf, agg_hbm.at[c].at[nsl])

    return k(xs2d, xd2d, row2, col2, attr_pt)


def _node_body(x_ref, agg_ref, wn_ref, bn_ref, out_ref):
    wn = wn_ref[...]
    agg = agg_ref[0] + agg_ref[1]
    out_ref[...] = (
        jnp.dot(x_ref[...], wn[0:D, :], preferred_element_type=jnp.float32)
        + jnp.dot(agg, wn[D:D + DE, :], preferred_element_type=jnp.float32)
        + bn_ref[...]
    )


def _node_update(x2d, agg, W_n, b_n2d):
    blk = 1000
    return pl.pallas_call(
        _node_body,
        grid=(N // blk,),
        in_specs=[
            pl.BlockSpec((blk, D), lambda i: (i, 0)),
            pl.BlockSpec((N_CORES, blk, DE), lambda i: (0, i, 0)),
            pl.BlockSpec((D + DE, D), lambda i: (0, 0)),
            pl.BlockSpec((1, D), lambda i: (0, 0)),
        ],
        out_specs=pl.BlockSpec((blk, D), lambda i: (i, 0)),
        out_shape=jax.ShapeDtypeStruct((N, D), jnp.float32),
    )(x2d, agg, W_n, b_n2d)


def kernel(x, edge_index, edge_attr, W_e, b_e, W_n, b_n):
    x2d = x[0]
    row2 = edge_index[0, 0].reshape(E // W, W)
    col2 = edge_index[0, 1].reshape(E // W, W)
    attr_t = edge_attr[0].T

    xs2d, xd2d, attr_pt = _pre(x2d, attr_t, W_e, b_e.reshape(DE, 1))
    eat, agg = _sc_edge_kernel(xs2d, xd2d, row2, col2, attr_pt)
    x_new = _node_update(x2d, agg, W_n, b_n.reshape(1, D))
    return (x_new[None], eat.T[None])


# speedup vs baseline: 24.0063x; 1.0578x over previous
import functools

import jax
import jax.numpy as jnp
from jax import lax
from jax.experimental import pallas as pl
from jax.experimental.pallas import tpu as pltpu
from jax.experimental.pallas import tpu_sc as plsc

N_CORES = 2
N_SUB = 16
NW = N_CORES * N_SUB

N = 10000
E = 320000
D = 128
DE = 16

W = 80
SUP = 2000
HSUP = SUP // 2
QSUP = SUP // 4
CPS = SUP // W
NSUP = (E // NW) // SUP
N_PAD = 10240
N_PER_SUB = N_PAD // N_SUB
T_PER_SUB = N // N_SUB
ZB = 64


def _pre_body(x_ref, at_ref, we_ref, be_ref, xs_ref, xd_ref, apt_ref):
    xb = x_ref[...]
    we = we_ref[...]
    xs_ref[...] = jnp.dot(xb, we[0:D, :], preferred_element_type=jnp.float32)
    xd_ref[...] = jnp.dot(xb, we[D:2 * D, :], preferred_element_type=jnp.float32)
    apt_ref[...] = lax.dot_general(
        we[2 * D:, :], at_ref[...],
        dimension_numbers=(((0,), (0,)), ((), ())),
        preferred_element_type=jnp.float32,
    ) + be_ref[...]


def _pre(x2d, attr_t, W_e, b_e_col):
    nblk = 1000
    ablk = E // (N // nblk)
    return pl.pallas_call(
        _pre_body,
        grid=(N // nblk,),
        in_specs=[
            pl.BlockSpec((nblk, D), lambda i: (i, 0)),
            pl.BlockSpec((DE, ablk), lambda i: (0, i)),
            pl.BlockSpec((2 * D + DE, DE), lambda i: (0, 0)),
            pl.BlockSpec((DE, 1), lambda i: (0, 0)),
        ],
        out_specs=[
            pl.BlockSpec((nblk, DE), lambda i: (i, 0)),
            pl.BlockSpec((nblk, DE), lambda i: (i, 0)),
            pl.BlockSpec((DE, ablk), lambda i: (0, i)),
        ],
        out_shape=[
            jax.ShapeDtypeStruct((N, DE), jnp.float32),
            jax.ShapeDtypeStruct((N, DE), jnp.float32),
            jax.ShapeDtypeStruct((DE, E), jnp.float32),
        ],
    )(x2d, attr_t, W_e, b_e_col)


def _sc_edge_kernel(xs2d, xd2d, row2, col2, attr_pt):
    mesh = plsc.VectorSubcoreMesh(core_axis_name="c", subcore_axis_name="s")

    @functools.partial(
        pl.kernel,
        out_type=(
            jax.ShapeDtypeStruct((DE, E), jnp.float32),
            jax.ShapeDtypeStruct((N_CORES, N_PAD, DE), jnp.float32),
        ),
        mesh=mesh,
        scratch_types=[
            pltpu.VMEM((CPS, W), jnp.int32),
            pltpu.VMEM((CPS, W), jnp.int32),
            pltpu.VMEM((SUP, DE), jnp.float32),
            pltpu.VMEM((SUP, DE), jnp.float32),
            pltpu.VMEM((DE, HSUP), jnp.float32),
            pltpu.VMEM((DE, HSUP), jnp.float32),
            pltpu.VMEM((ZB, DE), jnp.float32),
            pltpu.VMEM_SHARED((N_PAD, DE), jnp.float32),
            pltpu.SemaphoreType.DMA,
            pltpu.SemaphoreType.DMA,
            pltpu.SemaphoreType.DMA,
        ],
        compiler_params=pltpu.CompilerParams(
            use_tc_tiling_on_sc=False, needs_layout_passes=False),
    )
    def k(xs_hbm, xd_hbm, row_hbm, col_hbm, attr_hbm, ea_hbm, agg_hbm,
          row_v, col_v, src_v, dst_v, attr_v, eat_v, zbuf, agg_sp,
          isem, gsem, osem):
        c = lax.axis_index("c")
        s = lax.axis_index("s")
        wid = s * N_CORES + c
        lane = lax.iota(jnp.int32, DE)

        @pl.loop(0, ZB)
        def _(i):
            zbuf[i, :] = jnp.zeros((DE,), jnp.float32)

        @pl.loop(0, N_PER_SUB // ZB)
        def _(j):
            pltpu.sync_copy(
                zbuf, agg_sp.at[pl.ds(s * N_PER_SUB + j * ZB, ZB)])

        plsc.subcore_barrier()

        @pl.loop(0, NSUP)
        def _(ss):
            eoff = wid * (NSUP * SUP) + ss * SUP
            erow = eoff // W

            i1 = pltpu.async_copy(row_hbm.at[pl.ds(erow, CPS)], row_v, isem)
            i2 = pltpu.async_copy(col_hbm.at[pl.ds(erow, CPS)], col_v, isem)
            i3 = pltpu.async_copy(
                attr_hbm.at[pl.ds(0, DE), pl.ds(eoff, HSUP)], attr_v, isem)
            i1.wait(); i2.wait(); i3.wait()

            @pl.loop(0, 5)
            def _(g):
                for tt in range(5):
                    t = g * 5 + tt
                    dsl = pl.ds(t * W, W)
                    pltpu.async_copy(
                        xs_hbm.at[row_v.at[t]], src_v.at[dsl], gsem)
                    pltpu.async_copy(
                        xd_hbm.at[col_v.at[t]], dst_v.at[dsl], gsem)
                for tt in range(5):
                    t = g * 5 + tt
                    dsl = pl.ds(t * W, W)
                    pltpu.make_async_copy(
                        xs_hbm.at[row_v.at[t]], src_v.at[dsl], gsem).wait()
                    pltpu.make_async_copy(
                        xd_hbm.at[col_v.at[t]], dst_v.at[dsl], gsem).wait()

            for h in range(2):
                if h == 1:
                    pltpu.sync_copy(
                        attr_hbm.at[pl.ds(0, DE), pl.ds(eoff + HSUP, HSUP)],
                        attr_v)

                @pl.loop(0, HSUP)
                def _(i):
                    i2_ = h * HSUP + i
                    icol = jnp.full((DE,), i, jnp.int32)
                    av = plsc.load_gather(attr_v, [lane, icol])
                    ea = jnp.maximum(src_v[i2_, :] + dst_v[i2_, :] + av, 0.0)
                    src_v[i2_, :] = ea
                    plsc.store_scatter(eat_v, [lane, icol], ea)

                pltpu.sync_copy(
                    eat_v,
                    ea_hbm.at[pl.ds(0, DE), pl.ds(eoff + h * HSUP, HSUP)])

            @pl.loop(0, CPS)
            def _(t):
                dsl = pl.ds(t * W, W)
                pltpu.sync_copy(
                    src_v.at[dsl], agg_sp.at[col_v.at[t]], add=True)

        plsc.subcore_barrier()
        @pl.loop(0, N_PER_SUB // ZB)
        def _(j):
            nsl = pl.ds(s * N_PER_SUB + j * ZB, ZB)
            pltpu.sync_copy(agg_sp.at[nsl], zbuf)
            pltpu.sync_copy(zbuf, agg_hbm.at[c].at[nsl])

    return k(xs2d, xd2d, row2, col2, attr_pt)


def _node_body(x_ref, agg_ref, wn_ref, bn_ref, out_ref):
    wn = wn_ref[...]
    agg = agg_ref[0] + agg_ref[1]
    out_ref[...] = (
        jnp.dot(x_ref[...], wn[0:D, :], preferred_element_type=jnp.float32)
        + jnp.dot(agg, wn[D:D + DE, :], preferred_element_type=jnp.float32)
        + bn_ref[...]
    )


def _node_update(x2d, agg, W_n, b_n2d):
    blk = 1000
    return pl.pallas_call(
        _node_body,
        grid=(N // blk,),
        in_specs=[
            pl.BlockSpec((blk, D), lambda i: (i, 0)),
            pl.BlockSpec((N_CORES, blk, DE), lambda i: (0, i, 0)),
            pl.BlockSpec((D + DE, D), lambda i: (0, 0)),
            pl.BlockSpec((1, D), lambda i: (0, 0)),
        ],
        out_specs=pl.BlockSpec((blk, D), lambda i: (i, 0)),
        out_shape=jax.ShapeDtypeStruct((N, D), jnp.float32),
    )(x2d, agg, W_n, b_n2d)


def kernel(x, edge_index, edge_attr, W_e, b_e, W_n, b_n):
    x2d = x[0]
    row2 = edge_index[0, 0].reshape(E // W, W)
    col2 = edge_index[0, 1].reshape(E // W, W)
    attr_t = edge_attr[0].T

    xs2d, xd2d, attr_pt = _pre(x2d, attr_t, W_e, b_e.reshape(DE, 1))
    eat, agg = _sc_edge_kernel(xs2d, xd2d, row2, col2, attr_pt)
    x_new = _node_update(x2d, agg, W_n, b_n.reshape(1, D))
    return (x_new[None], eat.T[None])
